# baseline (speedup 1.0000x reference)
import functools

import jax
import jax.numpy as jnp
import numpy as np
from jax import lax
from jax.experimental import pallas as pl
from jax.experimental.pallas import tpu as pltpu
from jax.experimental.pallas import tpu_sc as plsc

_EPS = float(np.finfo(float).eps)
_LN2 = 0.6931471805599453
_SQRT2 = 1.4142135623730951
_NC, _NS, _L = 2, 16, 16
_NW = _NC * _NS


def _sc_log(x):
    xi = lax.bitcast_convert_type(x, jnp.int32)
    ex = lax.shift_right_logical(xi, 23) - 127
    mi = lax.bitwise_or(lax.bitwise_and(xi, 0x007FFFFF), 0x3F800000)
    m = lax.bitcast_convert_type(mi, jnp.float32)
    big = m > _SQRT2
    m = jnp.where(big, m * 0.5, m)
    ef = (ex + jnp.where(big, 1, 0)).astype(jnp.float32)
    s = (m - 1.0) / (m + 1.0)
    z = s * s
    poly = 2.0 + z * (2.0 / 3.0 + z * (0.4 + z * (2.0 / 7.0 + z * (2.0 / 9.0))))
    return ef * _LN2 + s * poly


def _make_sc_combine(E, B, PC):
    rows_pw = B // _NW
    mesh = plsc.VectorSubcoreMesh(
        core_axis_name="c", subcore_axis_name="s", num_cores=_NC, num_subcores=_NS
    )

    @functools.partial(
        pl.kernel,
        out_type=jax.ShapeDtypeStruct((B, PC), jnp.float32),
        mesh=mesh,
        scratch_types=[
            pltpu.VMEM((E, PC), jnp.float32),
            pltpu.VMEM((PC,), jnp.float32),
            pltpu.VMEM((rows_pw * E + _L,), jnp.float32),
        ],
    )
    def sc_combine(xs_hbm, g_hbm, o_hbm, xbuf, obuf, gbuf):
        c = lax.axis_index("c")
        s = lax.axis_index("s")
        wid = s * _NC + c
        base = wid * rows_pw
        pltpu.sync_copy(g_hbm.at[pl.ds(base * E, rows_pw * E)], gbuf.at[pl.ds(0, rows_pw * E)])

        def row_body(r, _):
            b = base + r
            for e in range(E):
                pltpu.sync_copy(xs_hbm.at[e, b], xbuf.at[e])
            gv = gbuf[pl.ds(r * E, _L)]
            gvecs = [jnp.full((_L,), gv[e], jnp.float32) for e in range(E)]

            def chunk(j, _):
                off = j * _L
                acc = gvecs[0] * jnp.exp(xbuf[0, pl.ds(off, _L)])
                for e in range(1, E):
                    acc = acc + gvecs[e] * jnp.exp(xbuf[e, pl.ds(off, _L)])
                acc = jnp.where(acc == 0.0, _EPS, acc)
                obuf[pl.ds(off, _L)] = _sc_log(acc)
                return 0

            lax.fori_loop(0, PC // _L, chunk, 0)
            pltpu.sync_copy(obuf, o_hbm.at[b])
            return 0

        lax.fori_loop(0, rows_pw, row_body, 0)

    return sc_combine


def kernel(xs, gates):
    E, B, P, C = xs.shape
    PC = P * C
    xs_f = xs.reshape(E, B, PC)
    gates_f = gates.reshape(B * E)
    out = _make_sc_combine(E, B, PC)(xs_f, gates_f)
    return out.reshape(B, P, C)


# speedup vs baseline: 2.2684x; 2.2684x over previous
import functools

import jax
import jax.numpy as jnp
import numpy as np
from jax import lax
from jax.experimental import pallas as pl
from jax.experimental.pallas import tpu as pltpu
from jax.experimental.pallas import tpu_sc as plsc

_EPS = float(np.finfo(float).eps)
_LN2 = 0.6931471805599453
_SQRT2 = 1.4142135623730951
_NC, _NS, _L = 2, 16, 16
_NW = _NC * _NS


def _sc_log(x):
    xi = lax.bitcast_convert_type(x, jnp.int32)
    ex = lax.shift_right_logical(xi, 23) - 127
    mi = lax.bitwise_or(lax.bitwise_and(xi, 0x007FFFFF), 0x3F800000)
    m = lax.bitcast_convert_type(mi, jnp.float32)
    big = m > _SQRT2
    m = jnp.where(big, m * 0.5, m)
    ef = (ex + jnp.where(big, 1, 0)).astype(jnp.float32)
    s = (m - 1.0) / (m + 1.0)
    z = s * s
    poly = 2.0 + z * (2.0 / 3.0 + z * (0.4 + z * (2.0 / 7.0 + z * (2.0 / 9.0))))
    return ef * _LN2 + s * poly


def _make_sc_combine(E, B, PC):
    rows_pw = B // _NW
    n_chunks = PC // _L
    mesh = plsc.VectorSubcoreMesh(
        core_axis_name="c", subcore_axis_name="s", num_cores=_NC, num_subcores=_NS
    )

    @functools.partial(
        pl.kernel,
        out_type=jax.ShapeDtypeStruct((B, PC), jnp.float32),
        mesh=mesh,
        scratch_types=[
            pltpu.VMEM((2, E, PC), jnp.float32),
            pltpu.VMEM((2, PC), jnp.float32),
            pltpu.VMEM((rows_pw * E + _L,), jnp.float32),
            pltpu.SemaphoreType.DMA,
            pltpu.SemaphoreType.DMA,
            pltpu.SemaphoreType.DMA,
            pltpu.SemaphoreType.DMA,
        ],
    )
    def sc_combine(xs_hbm, g_hbm, o_hbm, xbuf, obuf, gbuf, isem0, isem1, osem0, osem1):
        c = lax.axis_index("c")
        s = lax.axis_index("s")
        wid = s * _NC + c
        base = wid * rows_pw
        isems = (isem0, isem1)
        osems = (osem0, osem1)
        pltpu.sync_copy(
            g_hbm.at[pl.ds(base * E, rows_pw * E)], gbuf.at[pl.ds(0, rows_pw * E)]
        )
        for e in range(E):
            pltpu.async_copy(xs_hbm.at[e, base], xbuf.at[0, e], isem0)

        def pair_body(rr, _):
            for cur in range(2):
                nxt = 1 - cur
                r = rr * 2 + cur
                b = base + r

                @pl.when(r + 1 < rows_pw)
                def _prefetch():
                    for e in range(E):
                        pltpu.async_copy(xs_hbm.at[e, b + 1], xbuf.at[nxt, e], isems[nxt])

                for e in range(E):
                    pltpu.make_async_copy(xs_hbm.at[e, b], xbuf.at[cur, e], isems[cur]).wait()

                @pl.when(r >= 2)
                def _drain_store():
                    pltpu.make_async_copy(obuf.at[cur], o_hbm.at[b - 2], osems[cur]).wait()

                gv = gbuf[pl.ds(r * E, _L)]
                gvecs = [jnp.full((_L,), gv[e], jnp.float32) for e in range(E)]

                @plsc.parallel_loop(0, n_chunks, unroll=4)
                def _chunk(j):
                    off = j * _L
                    acc = gvecs[0] * jnp.exp(xbuf[cur, 0, pl.ds(off, _L)])
                    for e in range(1, E):
                        acc = acc + gvecs[e] * jnp.exp(xbuf[cur, e, pl.ds(off, _L)])
                    acc = jnp.where(acc == 0.0, _EPS, acc)
                    obuf[cur, pl.ds(off, _L)] = _sc_log(acc)

                pltpu.async_copy(obuf.at[cur], o_hbm.at[b], osems[cur])
            return 0

        lax.fori_loop(0, rows_pw // 2, pair_body, 0)
        pltpu.make_async_copy(obuf.at[0], o_hbm.at[base + rows_pw - 2], osem0).wait()
        pltpu.make_async_copy(obuf.at[1], o_hbm.at[base + rows_pw - 1], osem1).wait()

    return sc_combine


def kernel(xs, gates):
    E, B, P, C = xs.shape
    PC = P * C
    xs_f = xs.reshape(E, B, PC)
    gates_f = gates.reshape(B * E)
    out = _make_sc_combine(E, B, PC)(xs_f, gates_f)
    return out.reshape(B, P, C)


# speedup vs baseline: 3.1364x; 1.3826x over previous
import functools

import jax
import jax.numpy as jnp
import numpy as np
from jax import lax
from jax.experimental import pallas as pl
from jax.experimental.pallas import tpu as pltpu
from jax.experimental.pallas import tpu_sc as plsc

_EPS = float(np.finfo(float).eps)
_LN2 = 0.6931471805599453
_SQRT2 = 1.4142135623730951
_NC, _NS, _L = 2, 16, 16
_NW = _NC * _NS


def _sc_log(x):
    xi = lax.bitcast_convert_type(x, jnp.int32)
    ex = lax.shift_right_logical(xi, 23) - 127
    mi = lax.bitwise_or(lax.bitwise_and(xi, 0x007FFFFF), 0x3F800000)
    m = lax.bitcast_convert_type(mi, jnp.float32)
    big = m > _SQRT2
    m = jnp.where(big, m * 0.5, m)
    ef = (ex + jnp.where(big, 1, 0)).astype(jnp.float32)
    s = (m - 1.0) / (m + 1.0)
    z = s * s
    poly = 2.0 + z * (2.0 / 3.0 + z * (0.4 + z * (2.0 / 7.0 + z * (2.0 / 9.0))))
    return ef * _LN2 + s * poly


def _tc_body(x_ref, g_ref, o_ref):
    e_total = x_ref.shape[0]
    acc = jnp.exp(x_ref[0]) * g_ref[:, 0:1]
    for e in range(1, e_total):
        acc = acc + jnp.exp(x_ref[e]) * g_ref[:, e : e + 1]
    o_ref[...] = jnp.log(jnp.where(acc == 0.0, _EPS, acc))


def _tc_combine(xs_f, gates, n_rows, Bb=32):
    E, B, PC = xs_f.shape
    return pl.pallas_call(
        _tc_body,
        grid=(n_rows // Bb,),
        in_specs=[
            pl.BlockSpec((E, Bb, PC), lambda i: (0, i, 0)),
            pl.BlockSpec((Bb, E), lambda i: (i, 0)),
        ],
        out_specs=pl.BlockSpec((Bb, PC), lambda i: (i, 0)),
        out_shape=jax.ShapeDtypeStruct((n_rows, PC), jnp.float32),
    )(xs_f, gates)


def _make_sc_combine(E, B, PC, B0, n_rows):
    rows_pw = n_rows // _NW
    n_chunks = PC // _L
    mesh = plsc.VectorSubcoreMesh(
        core_axis_name="c", subcore_axis_name="s", num_cores=_NC, num_subcores=_NS
    )

    @functools.partial(
        pl.kernel,
        out_type=jax.ShapeDtypeStruct((n_rows, PC), jnp.float32),
        mesh=mesh,
        scratch_types=[
            pltpu.VMEM((2, E, PC), jnp.float32),
            pltpu.VMEM((2, PC), jnp.float32),
            pltpu.VMEM((rows_pw * E + _L,), jnp.float32),
            pltpu.SemaphoreType.DMA,
            pltpu.SemaphoreType.DMA,
            pltpu.SemaphoreType.DMA,
            pltpu.SemaphoreType.DMA,
        ],
    )
    def sc_combine(xs_hbm, g_hbm, o_hbm, xbuf, obuf, gbuf, isem0, isem1, osem0, osem1):
        c = lax.axis_index("c")
        s = lax.axis_index("s")
        wid = s * _NC + c
        base = wid * rows_pw
        bin0 = B0 + base
        isems = (isem0, isem1)
        osems = (osem0, osem1)
        pltpu.sync_copy(
            g_hbm.at[pl.ds(bin0 * E, rows_pw * E)], gbuf.at[pl.ds(0, rows_pw * E)]
        )
        for e in range(E):
            pltpu.async_copy(xs_hbm.at[e, bin0], xbuf.at[0, e], isem0)

        def pair_body(rr, _):
            for cur in range(2):
                nxt = 1 - cur
                r = rr * 2 + cur
                b = base + r
                bi = bin0 + r

                @pl.when(r + 1 < rows_pw)
                def _prefetch():
                    for e in range(E):
                        pltpu.async_copy(xs_hbm.at[e, bi + 1], xbuf.at[nxt, e], isems[nxt])

                for e in range(E):
                    pltpu.make_async_copy(xs_hbm.at[e, bi], xbuf.at[cur, e], isems[cur]).wait()

                @pl.when(r >= 2)
                def _drain_store():
                    pltpu.make_async_copy(obuf.at[cur], o_hbm.at[b - 2], osems[cur]).wait()

                gv = gbuf[pl.ds(r * E, _L)]
                gvecs = [jnp.full((_L,), gv[e], jnp.float32) for e in range(E)]

                @plsc.parallel_loop(0, n_chunks, unroll=4)
                def _chunk(j):
                    off = j * _L
                    acc = gvecs[0] * jnp.exp(xbuf[cur, 0, pl.ds(off, _L)])
                    for e in range(1, E):
                        acc = acc + gvecs[e] * jnp.exp(xbuf[cur, e, pl.ds(off, _L)])
                    acc = jnp.where(acc == 0.0, _EPS, acc)
                    obuf[cur, pl.ds(off, _L)] = _sc_log(acc)

                pltpu.async_copy(obuf.at[cur], o_hbm.at[b], osems[cur])
            return 0

        lax.fori_loop(0, rows_pw // 2, pair_body, 0)
        pltpu.make_async_copy(obuf.at[0], o_hbm.at[base + rows_pw - 2], osem0).wait()
        pltpu.make_async_copy(obuf.at[1], o_hbm.at[base + rows_pw - 1], osem1).wait()

    return sc_combine


_B_TC = 640


def kernel(xs, gates):
    E, B, P, C = xs.shape
    PC = P * C
    xs_f = xs.reshape(E, B, PC)
    gates_f = gates.reshape(B * E)
    n_sc = B - _B_TC
    out_tc = _tc_combine(xs_f, gates, _B_TC)
    out_sc = _make_sc_combine(E, B, PC, _B_TC, n_sc)(xs_f, gates_f)
    out = jnp.concatenate([out_tc, out_sc], axis=0)
    return out.reshape(B, P, C)


# speedup vs baseline: 8.8835x; 2.8324x over previous
import functools

import jax
import jax.numpy as jnp
import numpy as np
from jax import lax
from jax.experimental import pallas as pl
from jax.experimental.pallas import tpu as pltpu
from jax.experimental.pallas import tpu_sc as plsc

_EPS = float(np.finfo(float).eps)
_LN2 = 0.6931471805599453
_SQRT2 = 1.4142135623730951
_NC, _NS, _L = 2, 16, 16
_NW = _NC * _NS
_KK = 4


def _tc_body(x_ref, g_ref, o_ref):
    e_total = x_ref.shape[0]
    acc = jnp.exp(x_ref[0]) * g_ref[0][None, :]
    for e in range(1, e_total):
        acc = acc + jnp.exp(x_ref[e]) * g_ref[e][None, :]
    o_ref[...] = jnp.log(jnp.where(acc == 0.0, _EPS, acc))


def _tc_combine(xs_t, g_t, n_rows, PCb):
    E, PC, B = xs_t.shape
    return pl.pallas_call(
        _tc_body,
        grid=(n_rows // PCb,),
        in_specs=[
            pl.BlockSpec((E, PCb, B), lambda i: (0, i, 0)),
            pl.BlockSpec((E, B), lambda i: (0, 0)),
        ],
        out_specs=pl.BlockSpec((PCb, B), lambda i: (i, 0)),
        out_shape=jax.ShapeDtypeStruct((n_rows, B), jnp.float32),
    )(xs_t, g_t)


def _sc_log(x):
    xi = lax.bitcast_convert_type(x, jnp.int32)
    ex = lax.shift_right_logical(xi, 23) - 127
    mi = lax.bitwise_or(lax.bitwise_and(xi, 0x007FFFFF), 0x3F800000)
    m = lax.bitcast_convert_type(mi, jnp.float32)
    big = m > _SQRT2
    m = jnp.where(big, m * 0.5, m)
    ef = (ex + jnp.where(big, 1, 0)).astype(jnp.float32)
    s = (m - 1.0) / (m + 1.0)
    z = s * s
    poly = 2.0 + z * (2.0 / 3.0 + z * (0.4 + z * (2.0 / 7.0 + z * (2.0 / 9.0))))
    return ef * _LN2 + s * poly


def _make_sc_combine(E, PC, B, PC0, n_rows):
    rows_pw = n_rows // _NW
    n_slabs = rows_pw // _KK
    n_chunks = B // _L
    mesh = plsc.VectorSubcoreMesh(
        core_axis_name="c", subcore_axis_name="s", num_cores=_NC, num_subcores=_NS
    )

    @functools.partial(
        pl.kernel,
        out_type=jax.ShapeDtypeStruct((n_rows, B), jnp.float32),
        mesh=mesh,
        scratch_types=[
            pltpu.VMEM((2, E, _KK, B), jnp.float32),
            pltpu.VMEM((2, _KK, B), jnp.float32),
            pltpu.VMEM((E, B), jnp.float32),
            pltpu.SemaphoreType.DMA,
            pltpu.SemaphoreType.DMA,
            pltpu.SemaphoreType.DMA,
            pltpu.SemaphoreType.DMA,
        ],
    )
    def sc_combine(xs_hbm, g_hbm, o_hbm, xbuf, obuf, gbuf, isem0, isem1, osem0, osem1):
        c = lax.axis_index("c")
        s = lax.axis_index("s")
        wid = s * _NC + c
        base = wid * rows_pw
        pin0 = PC0 + base
        isems = (isem0, isem1)
        osems = (osem0, osem1)
        pltpu.sync_copy(g_hbm, gbuf)
        for e in range(E):
            pltpu.async_copy(xs_hbm.at[e, pl.ds(pin0, _KK)], xbuf.at[0, e], isem0)

        def pair_body(tt, _):
            for cur in range(2):
                nxt = 1 - cur
                t = tt * 2 + cur
                row_o = base + t * _KK
                row_i = pin0 + t * _KK

                @pl.when(t + 1 < n_slabs)
                def _prefetch():
                    for e in range(E):
                        pltpu.async_copy(
                            xs_hbm.at[e, pl.ds(row_i + _KK, _KK)],
                            xbuf.at[nxt, e],
                            isems[nxt],
                        )

                for e in range(E):
                    pltpu.make_async_copy(
                        xs_hbm.at[e, pl.ds(row_i, _KK)], xbuf.at[cur, e], isems[cur]
                    ).wait()

                @pl.when(t >= 2)
                def _drain_store():
                    pltpu.make_async_copy(
                        obuf.at[cur], o_hbm.at[pl.ds(row_o - 2 * _KK, _KK)], osems[cur]
                    ).wait()

                @plsc.parallel_loop(0, n_chunks, unroll=2)
                def _chunk(j):
                    off = j * _L
                    gv = [gbuf[e, pl.ds(off, _L)] for e in range(E)]
                    for kk in range(_KK):
                        acc = gv[0] * jnp.exp(xbuf[cur, 0, kk, pl.ds(off, _L)])
                        for e in range(1, E):
                            acc = acc + gv[e] * jnp.exp(xbuf[cur, e, kk, pl.ds(off, _L)])
                        acc = jnp.where(acc == 0.0, _EPS, acc)
                        obuf[cur, kk, pl.ds(off, _L)] = _sc_log(acc)

                pltpu.async_copy(obuf.at[cur], o_hbm.at[pl.ds(row_o, _KK)], osems[cur])
            return 0

        lax.fori_loop(0, n_slabs // 2, pair_body, 0)
        pltpu.make_async_copy(
            obuf.at[0], o_hbm.at[pl.ds(base + (n_slabs - 2) * _KK, _KK)], osem0
        ).wait()
        pltpu.make_async_copy(
            obuf.at[1], o_hbm.at[pl.ds(base + (n_slabs - 1) * _KK, _KK)], osem1
        ).wait()

    return sc_combine


_PC_TC = 4608


def kernel(xs, gates):
    E, B, P, C = xs.shape
    PC = P * C
    xs_t = jnp.transpose(xs, (0, 2, 3, 1)).reshape(E, PC, B)
    g_t = gates.T
    n_sc = PC - _PC_TC
    out_tc = _tc_combine(xs_t, g_t, _PC_TC, PCb=384)
    out_sc = _make_sc_combine(E, PC, B, _PC_TC, n_sc)(xs_t, g_t)
    out_t = jnp.concatenate([out_tc, out_sc], axis=0)
    return jnp.transpose(out_t.reshape(P, C, B), (2, 0, 1))


# speedup vs baseline: 14.3219x; 1.6122x over previous
import jax
import jax.numpy as jnp
import numpy as np
from jax.experimental import pallas as pl

_EPS = float(np.finfo(float).eps)


def _tc_body(x_ref, g_ref, o_ref):
    e_total = x_ref.shape[0]
    acc = jnp.exp(x_ref[0]) * g_ref[0][None, :]
    for e in range(1, e_total):
        acc = acc + jnp.exp(x_ref[e]) * g_ref[e][None, :]
    o_ref[...] = jnp.log(jnp.where(acc == 0.0, _EPS, acc))


def kernel(xs, gates):
    E, B, P, C = xs.shape
    PC = P * C
    xs_t = jnp.transpose(xs, (0, 2, 3, 1)).reshape(E, PC, B)
    g_t = gates.T
    PCb = 384

    out_t = pl.pallas_call(
        _tc_body,
        grid=(PC // PCb,),
        in_specs=[
            pl.BlockSpec((E, PCb, B), lambda i: (0, i, 0)),
            pl.BlockSpec((E, B), lambda i: (0, 0)),
        ],
        out_specs=pl.BlockSpec((PCb, B), lambda i: (i, 0)),
        out_shape=jax.ShapeDtypeStruct((PC, B), jnp.float32),
    )(xs_t, g_t)
    return jnp.transpose(out_t.reshape(P, C, B), (2, 0, 1))
